# baseline (speedup 1.0000x reference)
import functools

import jax
import jax.numpy as jnp
from jax import lax
from jax.experimental import pallas as pl
from jax.experimental.pallas import tpu as pltpu
from jax.experimental.pallas import tpu_sc as plsc

LANES = 128
PADW = 128


def _worker_id():
    info = plsc.get_sparse_core_info()
    return lax.axis_index("s") * info.num_cores + lax.axis_index("c")


GK = 4


@functools.cache
def _make_retile(V: int, D: int):
    info = plsc.get_sparse_core_info()
    NW = info.num_cores * info.num_subcores
    nfull = V // LANES
    rem = V - nfull * LANES
    GW = GK * LANES
    per_w = nfull // GK // NW
    n_extra = nfull - per_w * NW * GK
    assert per_w % 2 == 1 and 0 < n_extra <= GK

    mesh = plsc.VectorSubcoreMesh(core_axis_name="c", subcore_axis_name="s")

    @functools.partial(
        pl.kernel,
        out_type=jax.ShapeDtypeStruct((V, PADW), jnp.float32),
        mesh=mesh,
        compiler_params=pltpu.CompilerParams(use_tc_tiling_on_sc=True,
                                             needs_layout_passes=False),
        scratch_types=[
            pltpu.VMEM((2, D, GW), jnp.float32),
            pltpu.VMEM((2, LANES, PADW + 1), jnp.float32),
            pltpu.SemaphoreType.DMA((2,)),
            pltpu.SemaphoreType.DMA((2,)),
        ],
    )
    def retile_kernel(tblT_hbm, tail_hbm, out_hbm, ibuf, obuf, rsem, wsem):
        wid = _worker_id()
        g0 = per_w * wid
        lane_iota = lax.iota(jnp.int32, 16)
        c_groups = [lane_iota + 16 * g for g in range(LANES // 16)]

        def rstart(g, b):
            for tr in range(D // 8):
                pltpu.async_copy(
                    tblT_hbm.at[pl.ds(8 * tr, 8), pl.ds(g * GW, GW)],
                    ibuf.at[b, pl.ds(8 * tr, 8)], rsem.at[b])

        def rwait(b):
            for tr in range(D // 8):
                pltpu.make_async_copy(
                    tblT_hbm.at[pl.ds(0, 8), pl.ds(0, GW)],
                    ibuf.at[b, pl.ds(8 * tr, 8)], rsem.at[b]).wait()

        def wstart(blk, jb):
            pltpu.async_copy(obuf.at[jb, pl.ds(0, LANES), pl.ds(0, PADW)],
                             out_hbm.at[pl.ds(blk * LANES, LANES)],
                             wsem.at[jb])

        def wwait(jb):
            pltpu.make_async_copy(obuf.at[jb, pl.ds(0, LANES), pl.ds(0, PADW)],
                                  out_hbm.at[pl.ds(0, LANES)],
                                  wsem.at[jb]).wait()

        def transpose(b, j, jb):
            def grp(i, _):
                for dd in range(2):
                    d = i * 2 + dd
                    dv = jnp.full((16,), d, jnp.int32)
                    xs = [ibuf[b, d, pl.ds(j * LANES + cg * 16, 16)]
                          for cg in range(LANES // 16)]
                    for cg in range(LANES // 16):
                        plsc.store_scatter(obuf.at[jb], [c_groups[cg], dv],
                                           xs[cg])
                return 0
            lax.fori_loop(0, D // 2, grp, 0, unroll=False)

        def do_group(g, b, first):
            rwait(b)
            for j in range(GK):
                jb = j % 2
                if not (first and j < 2):
                    wwait(jb)
                transpose(b, j, jb)
                wstart(g * GK + j, jb)

            @pl.when(g + 2 < g0 + per_w)
            def _():
                rstart(g + 2, b)

        rstart(g0, 0)
        rstart(g0 + 1, 1)
        do_group(g0, 0, True)

        def step(i, _):
            g = g0 + 1 + 2 * i
            do_group(g, 1, False)
            do_group(g + 1, 0, False)
            return 0

        lax.fori_loop(0, (per_w - 1) // 2, step, 0)
        wwait(0)
        wwait(1)

        @pl.when(wid == NW - 1)
        def _():
            ge = per_w * NW
            rstart(ge, 0)
            rwait(0)
            for j in range(n_extra):
                jb = j % 2
                transpose(0, j, jb)
                pltpu.sync_copy(obuf.at[jb, pl.ds(0, LANES), pl.ds(0, PADW)],
                                out_hbm.at[pl.ds((ge * GK + j) * LANES,
                                                 LANES)])
            if rem:
                pltpu.sync_copy(tail_hbm, ibuf.at[0, pl.ds(0, rem),
                                                  pl.ds(0, PADW)])
                pltpu.sync_copy(ibuf.at[0, pl.ds(0, rem), pl.ds(0, PADW)],
                                out_hbm.at[pl.ds(nfull * LANES, rem)])

    return retile_kernel


@functools.cache
def _make_gather(V: int, D: int, SQ: int, ST: int, B: int):
    info = plsc.get_sparse_core_info()
    NW = info.num_cores * info.num_subcores
    assert B // LANES == NW and SQ % 2 == 0 and ST % 2 == 0

    SQ8 = (SQ + 7) // 8
    ST8 = (ST + 7) // 8
    q_row0 = 0
    t_row0 = SQ8 * 8
    n_rows = t_row0 + ST8 * 8

    mesh = plsc.VectorSubcoreMesh(core_axis_name="c", subcore_axis_name="s")

    @functools.partial(
        pl.kernel,
        out_type=(
            jax.ShapeDtypeStruct((SQ, D, B), jnp.float32),
            jax.ShapeDtypeStruct((ST, D, B), jnp.float32),
        ),
        mesh=mesh,
        compiler_params=pltpu.CompilerParams(use_tc_tiling_on_sc=True,
                                             needs_layout_passes=False),
        scratch_types=[
            pltpu.VMEM((n_rows, LANES), jnp.int32),
            pltpu.VMEM((2, LANES, PADW), jnp.float32),
            pltpu.VMEM((2, D, LANES + 1), jnp.float32),
            pltpu.SemaphoreType.DMA((2,)),
            pltpu.SemaphoreType.DMA((2,)),
        ],
    )
    def gather_kernel(table_hbm, qT_hbm, tT_hbm, out_q, out_t,
                      idx_v, gbuf, tbuf, gsem, wsem):
        wid = _worker_id()
        col0 = wid * LANES
        tok_iota = lax.iota(jnp.int32, 16)

        for arr, row0, S8 in ((qT_hbm, q_row0, SQ8), (tT_hbm, t_row0, ST8)):
            for st in range(S8):
                pltpu.sync_copy(
                    arr.at[pl.ds(8 * st, 8), pl.ds(col0, LANES)],
                    idx_v.at[pl.ds(row0 + 8 * st, 8)])

        d_groups = [tok_iota + 16 * g for g in range(D // 16)]

        def transpose(b):
            def grp(i, _):
                for tt in range(2):
                    t0 = i * 4 + tt * 2
                    tvs = [jnp.full((16,), t0 + j, jnp.int32) for j in range(2)]
                    xs = [gbuf[b, t0 + j, pl.ds(dg * 16, 16)]
                          for j in range(2) for dg in range(D // 16)]
                    for j in range(2):
                        for dg in range(D // 16):
                            plsc.store_scatter(tbuf.at[b],
                                               [d_groups[dg], tvs[j]],
                                               xs[j * (D // 16) + dg])
                return 0
            lax.fori_loop(0, LANES // 4, grp, 0, unroll=False)

        def run(out_ref, row0, n_units):
            def gstart(u, b):
                pltpu.async_copy(table_hbm.at[idx_v.at[row0 + u]],
                                 gbuf.at[b], gsem.at[b])

            def gwait(b):
                pltpu.make_async_copy(table_hbm.at[idx_v.at[0]],
                                      gbuf.at[b], gsem.at[b]).wait()

            def wstart(u, b):
                pltpu.async_copy(
                    tbuf.at[b, pl.ds(0, D), pl.ds(0, LANES)],
                    out_ref.at[u, pl.ds(0, D), pl.ds(col0, LANES)],
                    wsem.at[b])

            def wwait(b):
                pltpu.make_async_copy(
                    tbuf.at[b, pl.ds(0, D), pl.ds(0, LANES)],
                    out_ref.at[0, pl.ds(0, D), pl.ds(0, LANES)],
                    wsem.at[b]).wait()

            gstart(0, 0)
            gstart(1, 1)

            def step(i, _):
                u0 = 2 * i
                for b in range(2):
                    gwait(b)

                    @pl.when(i > 0)
                    def _():
                        wwait(b)

                    transpose(b)
                    wstart(u0 + b, b)

                    @pl.when(u0 + 2 + b < n_units)
                    def _():
                        gstart(u0 + 2 + b, b)
                return 0

            lax.fori_loop(0, n_units // 2, step, 0)
            wwait(0)
            wwait(1)

        run(out_q, q_row0, SQ)
        run(out_t, t_row0, ST)

    return gather_kernel


def kernel(table, query, title):
    V, D = table.shape
    B, SQ = query.shape
    _, ST = title.shape
    rem = V % LANES
    tblT = jnp.transpose(table)
    tail = jnp.pad(table[V - rem:, :], ((0, 0), (0, PADW - D)))
    qT = jnp.transpose(query.astype(jnp.int32))
    tT = jnp.transpose(title.astype(jnp.int32))
    qT = jnp.pad(qT, ((0, -SQ % 8), (0, 0)))
    tT = jnp.pad(tT, ((0, -ST % 8), (0, 0)))
    table128 = _make_retile(V, D)(tblT, tail)
    out_qT, out_tT = _make_gather(V, D, SQ, ST, B)(table128, qT, tT)
    return (jnp.transpose(out_qT, (2, 0, 1)), jnp.transpose(out_tT, (2, 0, 1)))


# speedup vs baseline: 1.9156x; 1.9156x over previous
import functools

import jax
import jax.numpy as jnp
from jax import lax
from jax.experimental import pallas as pl
from jax.experimental.pallas import tpu as pltpu
from jax.experimental.pallas import tpu_sc as plsc

CH = 128


@functools.cache
def _make_gather(n_q: int, n_t: int, dim: int):
    info = plsc.get_sparse_core_info()
    NC, NS = info.num_cores, info.num_subcores
    NW = NC * NS
    n_all = n_q + n_t
    assert n_q % CH == 0 and n_all % (NW * CH) == 0
    n_ch = n_all // (NW * CH)
    per = n_ch * CH
    q_chunks = n_q // CH
    NBUF = 10
    assert n_ch % NBUF == 0
    ngroups = n_ch // NBUF

    mesh = plsc.VectorSubcoreMesh(core_axis_name="c", subcore_axis_name="s")

    @functools.partial(
        pl.kernel,
        out_type=(
            jax.ShapeDtypeStruct((n_q, dim), jnp.float32),
            jax.ShapeDtypeStruct((n_t, dim), jnp.float32),
        ),
        mesh=mesh,
        compiler_params=pltpu.CompilerParams(use_tc_tiling_on_sc=False),
        scratch_types=[
            pltpu.VMEM((per,), jnp.int32),
            pltpu.VMEM((NBUF, CH, dim), jnp.float32),
            pltpu.SemaphoreType.DMA((NBUF,)),
            pltpu.SemaphoreType.DMA((NBUF,)),
        ],
    )
    def gather_kernel(table_hbm, idx_hbm, out_q, out_t,
                      idx_v, rows_v, gsem, wsem):
        c = lax.axis_index("c")
        s = lax.axis_index("s")
        wid = s * NC + c

        pltpu.sync_copy(idx_hbm.at[pl.ds(wid * per, per)], idx_v)

        def gstart(b, j):
            pltpu.async_copy(
                table_hbm.at[idx_v.at[pl.ds(j * CH, CH)]],
                rows_v.at[b], gsem.at[b])

        def gwait(b):
            pltpu.make_async_copy(
                table_hbm.at[idx_v.at[pl.ds(0, CH)]],
                rows_v.at[b], gsem.at[b]).wait()

        def wstart(b, j):
            g = wid * n_ch + j

            @pl.when(g < q_chunks)
            def _():
                pltpu.async_copy(rows_v.at[b],
                                 out_q.at[pl.ds(g * CH, CH)], wsem.at[b])

            @pl.when(g >= q_chunks)
            def _():
                pltpu.async_copy(rows_v.at[b],
                                 out_t.at[pl.ds((g - q_chunks) * CH, CH)],
                                 wsem.at[b])

        def wwait(b):
            pltpu.make_async_copy(rows_v.at[b],
                                  out_q.at[pl.ds(0, CH)], wsem.at[b]).wait()

        for b in range(NBUF):
            gstart(b, b)

        def group(i, _):
            j0 = i * NBUF
            for b in range(NBUF):
                gwait(b)
                wstart(b, j0 + b)
            for b in range(NBUF):
                wwait(b)
                gstart(b, j0 + NBUF + b)
            return 0

        lax.fori_loop(0, ngroups - 1, group, 0)
        j0 = (ngroups - 1) * NBUF
        for b in range(NBUF):
            gwait(b)
            wstart(b, j0 + b)
        for b in range(NBUF):
            wwait(b)

    return gather_kernel


def kernel(table, query, title):
    n_q = query.size
    n_t = title.size
    idx_all = jnp.concatenate([
        query.astype(jnp.int32).reshape(n_q),
        title.astype(jnp.int32).reshape(n_t),
    ])
    fn = _make_gather(n_q, n_t, table.shape[1])
    out_q, out_t = fn(table, idx_all)
    return (out_q.reshape(*query.shape, table.shape[1]),
            out_t.reshape(*title.shape, table.shape[1]))
